# baseline (speedup 1.0000x reference)
import functools

import jax
import jax.numpy as jnp
from jax import lax
from jax.experimental import pallas as pl
from jax.experimental.pallas import tpu as pltpu
from jax.experimental.pallas import tpu_sc as plsc

NN = 10000
D = 128
NE = 320000

NC = 2
NS = 16
L = 16
NW = NC * NS
NPT = 320
NF = D // L
CH = 8000
NCHUNK = NE // CH
GK = 128
GR = 80


def _mm_body(x_ref, wd_ref, w2p_ref, b_ref, a_ref, bb_ref):
    x = x_ref[...]
    a_ref[...] = jnp.dot(x, wd_ref[...], preferred_element_type=jnp.float32) + b_ref[0:1, :]
    bb = jnp.dot(x, w2p_ref[...], preferred_element_type=jnp.float32)
    bb_ref[...] = bb.astype(jnp.bfloat16)


def _precompute(x, Wd, W2p, b8):
    return pl.pallas_call(
        _mm_body,
        grid=(10,),
        in_specs=[
            pl.BlockSpec((1000, D), lambda i: (i, 0)),
            pl.BlockSpec((D, D), lambda i: (0, 0)),
            pl.BlockSpec((D, D), lambda i: (0, 0)),
            pl.BlockSpec((8, D), lambda i: (0, 0)),
        ],
        out_specs=[
            pl.BlockSpec((1000, D), lambda i: (i, 0)),
            pl.BlockSpec((1000, D), lambda i: (i, 0)),
        ],
        out_shape=[
            jax.ShapeDtypeStruct((NN, D), jnp.float32),
            jax.ShapeDtypeStruct((NN, D), jnp.bfloat16),
        ],
    )(x, Wd, W2p, b8)


@functools.partial(
    pl.kernel,
    out_type=jax.ShapeDtypeStruct((NN, D), jnp.float32),
    mesh=plsc.VectorSubcoreMesh(core_axis_name="c", subcore_axis_name="s"),
    compiler_params=pltpu.CompilerParams(needs_layout_passes=False),
    scratch_types=[
        [pltpu.VMEM(((NPT + 1) * L,), jnp.int32) for _ in range(NF // 2)],
        pltpu.VMEM((CH,), jnp.int32),
        pltpu.VMEM((CH,), jnp.int32),
        pltpu.VMEM((CH,), jnp.int32),
        pltpu.VMEM((CH,), jnp.int32),
        pltpu.VMEM((CH + GK,), jnp.int32),
        pltpu.VMEM((CH + GK,), jnp.int32),
        pltpu.VMEM((GK, D), jnp.int32),
        pltpu.VMEM((GK, D), jnp.int32),
        pltpu.VMEM((GR, D), jnp.float32),
        pltpu.VMEM((GR, D), jnp.float32),
        pltpu.SemaphoreType.DMA,
        pltpu.SemaphoreType.DMA,
        pltpu.SemaphoreType.DMA,
        pltpu.SemaphoreType.DMA,
        pltpu.SemaphoreType.DMA,
        pltpu.SemaphoreType.DMA,
    ],
)
def _sc_segmax(a_hbm, b_hbm, dst_hbm, src_hbm, out_hbm,
               cbufs, dstb_a, dstb_b, srcb_a, srcb_b, ldb, srcc,
               brows0, brows1, fbuf0, fbuf1,
               semd_a, semd_b, sems_a, sems_b, semg0, semg1):
    wid = lax.axis_index("s") * NC + lax.axis_index("c")
    lo = wid * NPT

    neg = jnp.full((L,), jnp.uint32(0xFF80FF80).astype(jnp.int32), jnp.int32)

    def init_body(i, _):
        for f in range(NF // 2):
            cbufs[f][pl.ds(i * L, L)] = neg
        return 0

    lax.fori_loop(0, NPT + 1, init_body, 0, unroll=2)

    padv = jnp.full((L,), NPT * L, jnp.int32)
    zv = jnp.zeros((L,), jnp.int32)

    def load_desc(ci, dstb, srcb, semd, sems):
        dd = pltpu.make_async_copy(dst_hbm.at[pl.ds(ci * CH, CH)], dstb, semd)
        ss = pltpu.make_async_copy(src_hbm.at[pl.ds(ci * CH, CH)], srcb, sems)
        return dd, ss

    def gather_desc(k0, brows, semg):
        return pltpu.make_async_copy(b_hbm.at[srcc.at[pl.ds(k0, GK)]], brows, semg)

    hi_mask = jnp.full((L,), jnp.uint32(0xFFFF0000).astype(jnp.int32), jnp.int32)

    def update(k0, brows):
        def egroup_body(eg, _):
            bases = ldb[pl.ds(k0 + eg * L, L)]
            for j in range(L):
                base = bases[j]
                row = eg * L + j
                sl = pl.ds(base, L)
                for f in range(NF // 2):
                    bw = brows[row, pl.ds(f * L, L)]
                    b0 = plsc.bitcast(bw << 16, jnp.float32)
                    b1 = plsc.bitcast(bw & hi_mask, jnp.float32)
                    cb = cbufs[f]
                    cw = cb[sl]
                    c0 = plsc.bitcast(cw << 16, jnp.float32)
                    c1 = plsc.bitcast(cw & hi_mask, jnp.float32)
                    m0 = plsc.bitcast(jnp.maximum(c0, b0), jnp.int32)
                    m1 = plsc.bitcast(jnp.maximum(c1, b1), jnp.int32)
                    cb[sl] = jnp.bitwise_or(
                        lax.shift_right_logical(m0, 16), m1 & hi_mask)
            return 0

        with jax.named_scope("phase_upd"):
            lax.fori_loop(0, GK // L, egroup_body, 0)

    def process_chunk(m, dstb, srcb):
        def filt(gi, mm):
            d = dstb[pl.ds(gi * L, L)]
            s = srcb[pl.ds(gi * L, L)]
            t = d - lo
            msk = plsc.bitcast(t, jnp.uint32) < jnp.uint32(NPT)
            cs = plsc.cumsum(msk.astype(jnp.int32))
            pos = mm + cs - 1
            plsc.store_scatter(ldb, [pos], t * L, mask=msk)
            plsc.store_scatter(srcc, [pos], s, mask=msk)
            return mm + cs[L - 1]

        with jax.named_scope("phase_filt"):
            m = plsc.parallel_loop(0, CH // L, carry=m, unroll=8)(filt)

        nr = m >> 7

        def pair_round(rp, _):
            r0 = rp * 2

            @pl.when(r0 + 1 < nr)
            def _():
                gather_desc((r0 + 1) * GK, brows1, semg1).start()

            with jax.named_scope("phase_gwait"):
                gather_desc(r0 * GK, brows0, semg0).wait()
            update(r0 * GK, brows0)

            @pl.when(r0 + 2 < nr)
            def _():
                gather_desc((r0 + 2) * GK, brows0, semg0).start()

            @pl.when(r0 + 1 < nr)
            def _():
                with jax.named_scope("phase_gwait"):
                    gather_desc((r0 + 1) * GK, brows1, semg1).wait()
                update((r0 + 1) * GK, brows1)

            return 0

        with jax.named_scope("phase_rounds"):
            @pl.when(nr > 0)
            def _():
                gather_desc(0, brows0, semg0).start()

            lax.fori_loop(0, (nr + 1) >> 1, pair_round, 0)

            @pl.when(nr > 0)
            def _():
                for j in range(GK // L):
                    lv = ldb[pl.ds((nr << 7) + j * L, L)]
                    sv = srcc[pl.ds((nr << 7) + j * L, L)]
                    ldb[pl.ds(j * L, L)] = lv
                    srcc[pl.ds(j * L, L)] = sv

        return m - (nr << 7)

    da0, sa0 = load_desc(0, dstb_a, srcb_a, semd_a, sems_a)
    da0.start()
    sa0.start()

    def chunk_pair(cp, m):
        c0 = cp * 2
        db, sb = load_desc(c0 + 1, dstb_b, srcb_b, semd_b, sems_b)
        db.start()
        sb.start()
        da, sa = load_desc(c0, dstb_a, srcb_a, semd_a, sems_a)
        with jax.named_scope("phase_ldwait"):
            da.wait()
            sa.wait()
        m = process_chunk(m, dstb_a, srcb_a)

        @pl.when(c0 + 2 < NCHUNK)
        def _():
            da2, sa2 = load_desc(c0 + 2, dstb_a, srcb_a, semd_a, sems_a)
            da2.start()
            sa2.start()

        with jax.named_scope("phase_ldwait"):
            db.wait()
            sb.wait()
        m = process_chunk(m, dstb_b, srcb_b)
        return m

    m = lax.fori_loop(0, NCHUNK // 2, chunk_pair, 0)

    for j in range(GK // L):
        ldb[pl.ds(m + j * L, L)] = padv
        srcc[pl.ds(m + j * L, L)] = zv

    @pl.when(m > 0)
    def _():
        gather_desc(0, brows0, semg0).start()
        gather_desc(0, brows0, semg0).wait()
        update(0, brows0)

    ng = jnp.minimum(NPT, NN - lo) // GR
    fslots = (fbuf0, fbuf1)
    fsems = (semg0, semg1)

    def fin_desc(g, slot):
        return pltpu.make_async_copy(
            a_hbm.at[pl.ds(lo + g * GR, GR)], fslots[slot], fsems[slot])

    with jax.named_scope("phase_fin"):
        fin_desc(0, 0).start()
        for g in range(NPT // GR):
            @pl.when(g < ng)
            def _(g=g):
                abuf = fslots[g % 2]
                fin_desc(g, g % 2).wait()
                if g + 1 < NPT // GR:
                    @pl.when(g + 1 < ng)
                    def _():
                        fin_desc(g + 1, (g + 1) % 2).start()

                def row_fin(r, _):
                    row0 = g * GR
                    for f in range(NF // 2):
                        cw = cbufs[f][pl.ds((row0 + r) * L, L)]
                        c0 = plsc.bitcast(cw << 16, jnp.float32)
                        c1 = plsc.bitcast(cw & hi_mask, jnp.float32)
                        s0 = pl.ds(f * 2 * L, L)
                        s1 = pl.ds(f * 2 * L + L, L)
                        abuf[r, s0] = jnp.maximum(abuf[r, s0] + c0, 0.0)
                        abuf[r, s1] = jnp.maximum(abuf[r, s1] + c1, 0.0)
                    return 0

                lax.fori_loop(0, GR, row_fin, 0, unroll=2)
                pltpu.sync_copy(abuf, out_hbm.at[pl.ds(lo + g * GR, GR)])

    return None


_PERM = [0] * D
for _f in range(D // 32):
    for _w in range(16):
        _PERM[_f * 32 + 2 * _w] = _f * 32 + _w
        _PERM[_f * 32 + 2 * _w + 1] = _f * 32 + 16 + _w


def kernel(x, edge_index, W, b):
    b8 = jnp.broadcast_to(b.reshape(1, D), (8, D))
    Wd = W[:D, :] - W[D:, :]
    W2p = W[D:, :][:, jnp.array(_PERM, jnp.int32)]
    a, bbp16 = _precompute(x, Wd, W2p, b8)
    bbp = jax.lax.bitcast_convert_type(bbp16.reshape(NN, D // 2, 2), jnp.int32)
    bbp = jnp.concatenate([bbp, jnp.zeros((NN, D // 2), jnp.int32)], axis=1)
    src = edge_index[0]
    dst = edge_index[1]
    return _sc_segmax(a, bbp, dst, src)


# speedup vs baseline: 1.0256x; 1.0256x over previous
import functools

import jax
import jax.numpy as jnp
from jax import lax
from jax.experimental import pallas as pl
from jax.experimental.pallas import tpu as pltpu
from jax.experimental.pallas import tpu_sc as plsc

NN = 10000
D = 128
NE = 320000

NC = 2
NS = 16
L = 16
NW = NC * NS
NPT = 320
NF = D // L
CH = 8000
NCHUNK = NE // CH
GK = 128
GR = 80


def _mm_body(x_ref, w_ref, b_ref, a_ref, bb_ref):
    x = x_ref[...]
    w1 = w_ref[:D, :]
    w2 = w_ref[D:, :]
    bb = jnp.dot(x, w2, preferred_element_type=jnp.float32)
    bb_ref[...] = bb
    a_ref[...] = jnp.dot(x, w1 - w2, preferred_element_type=jnp.float32) + b_ref[0:1, :]


def _precompute(x, W, b8):
    return pl.pallas_call(
        _mm_body,
        grid=(10,),
        in_specs=[
            pl.BlockSpec((1000, D), lambda i: (i, 0)),
            pl.BlockSpec((2 * D, D), lambda i: (0, 0)),
            pl.BlockSpec((8, D), lambda i: (0, 0)),
        ],
        out_specs=[
            pl.BlockSpec((1000, D), lambda i: (i, 0)),
            pl.BlockSpec((1000, D), lambda i: (i, 0)),
        ],
        out_shape=[
            jax.ShapeDtypeStruct((NN, D), jnp.float32),
            jax.ShapeDtypeStruct((NN, D), jnp.float32),
        ],
    )(x, W, b8)


@functools.partial(
    pl.kernel,
    out_type=jax.ShapeDtypeStruct((NN, D), jnp.float32),
    mesh=plsc.VectorSubcoreMesh(core_axis_name="c", subcore_axis_name="s"),
    compiler_params=pltpu.CompilerParams(needs_layout_passes=False),
    scratch_types=[
        [pltpu.VMEM(((NPT + 1) * L,), jnp.int32) for _ in range(NF // 2)],
        pltpu.VMEM((CH,), jnp.int32),
        pltpu.VMEM((CH,), jnp.int32),
        pltpu.VMEM((CH,), jnp.int32),
        pltpu.VMEM((CH,), jnp.int32),
        pltpu.VMEM((CH + GK,), jnp.int32),
        pltpu.VMEM((CH + GK,), jnp.int32),
        pltpu.VMEM((GK, D), jnp.float32),
        pltpu.VMEM((GK, D), jnp.float32),
        pltpu.SemaphoreType.DMA,
        pltpu.SemaphoreType.DMA,
        pltpu.SemaphoreType.DMA,
        pltpu.SemaphoreType.DMA,
        pltpu.SemaphoreType.DMA,
        pltpu.SemaphoreType.DMA,
    ],
)
def _sc_segmax(a_hbm, b_hbm, dst_hbm, src_hbm, out_hbm,
               cbufs, dstb_a, dstb_b, srcb_a, srcb_b, ldb, srcc,
               brows0, brows1, semd_a, semd_b, sems_a, sems_b, semg0, semg1):
    wid = lax.axis_index("s") * NC + lax.axis_index("c")
    lo = wid * NPT

    neg = jnp.full((L,), jnp.uint32(0xFF80FF80).astype(jnp.int32), jnp.int32)

    def init_body(i, _):
        for f in range(NF // 2):
            cbufs[f][pl.ds(i * L, L)] = neg
        return 0

    lax.fori_loop(0, NPT + 1, init_body, 0, unroll=2)

    padv = jnp.full((L,), NPT * L, jnp.int32)
    zv = jnp.zeros((L,), jnp.int32)

    def load_desc(ci, dstb, srcb, semd, sems):
        dd = pltpu.make_async_copy(dst_hbm.at[pl.ds(ci * CH, CH)], dstb, semd)
        ss = pltpu.make_async_copy(src_hbm.at[pl.ds(ci * CH, CH)], srcb, sems)
        return dd, ss

    def gather_desc(k0, brows, semg):
        return pltpu.make_async_copy(b_hbm.at[srcc.at[pl.ds(k0, GK)]], brows, semg)

    hi_mask = jnp.full((L,), jnp.uint32(0xFFFF0000).astype(jnp.int32), jnp.int32)

    def update(k0, brows):
        def egroup_body(eg, _):
            bases = ldb[pl.ds(k0 + eg * L, L)]
            for j in range(L):
                base = bases[j]
                row = eg * L + j
                sl = pl.ds(base, L)
                for f in range(NF // 2):
                    b0 = brows[row, pl.ds(f * 2 * L, L)]
                    b1 = brows[row, pl.ds(f * 2 * L + L, L)]
                    cb = cbufs[f]
                    cw = cb[sl]
                    c0 = plsc.bitcast(cw << 16, jnp.float32)
                    c1 = plsc.bitcast(cw & hi_mask, jnp.float32)
                    m0 = plsc.bitcast(jnp.maximum(c0, b0), jnp.int32)
                    m1 = plsc.bitcast(jnp.maximum(c1, b1), jnp.int32)
                    cb[sl] = jnp.bitwise_or(
                        lax.shift_right_logical(m0, 16), m1 & hi_mask)
            return 0

        with jax.named_scope("phase_upd"):
            lax.fori_loop(0, GK // L, egroup_body, 0, unroll=2)

    def process_chunk(m, dstb, srcb):
        def filt(gi, mm):
            d = dstb[pl.ds(gi * L, L)]
            s = srcb[pl.ds(gi * L, L)]
            t = d - lo
            msk = plsc.bitcast(t, jnp.uint32) < jnp.uint32(NPT)
            cs = plsc.cumsum(msk.astype(jnp.int32))
            pos = mm + cs - 1
            plsc.store_scatter(ldb, [pos], t * L, mask=msk)
            plsc.store_scatter(srcc, [pos], s, mask=msk)
            return mm + cs[L - 1]

        with jax.named_scope("phase_filt"):
            m = plsc.parallel_loop(0, CH // L, carry=m, unroll=8)(filt)

        nr = m >> 7

        def pair_round(rp, _):
            r0 = rp * 2

            @pl.when(r0 + 1 < nr)
            def _():
                gather_desc((r0 + 1) * GK, brows1, semg1).start()

            with jax.named_scope("phase_gwait"):
                gather_desc(r0 * GK, brows0, semg0).wait()
            update(r0 * GK, brows0)

            @pl.when(r0 + 2 < nr)
            def _():
                gather_desc((r0 + 2) * GK, brows0, semg0).start()

            @pl.when(r0 + 1 < nr)
            def _():
                with jax.named_scope("phase_gwait"):
                    gather_desc((r0 + 1) * GK, brows1, semg1).wait()
                update((r0 + 1) * GK, brows1)

            return 0

        with jax.named_scope("phase_rounds"):
            @pl.when(nr > 0)
            def _():
                gather_desc(0, brows0, semg0).start()

            lax.fori_loop(0, (nr + 1) >> 1, pair_round, 0)

            @pl.when(nr > 0)
            def _():
                for j in range(GK // L):
                    lv = ldb[pl.ds((nr << 7) + j * L, L)]
                    sv = srcc[pl.ds((nr << 7) + j * L, L)]
                    ldb[pl.ds(j * L, L)] = lv
                    srcc[pl.ds(j * L, L)] = sv

        return m - (nr << 7)

    da0, sa0 = load_desc(0, dstb_a, srcb_a, semd_a, sems_a)
    da0.start()
    sa0.start()

    def chunk_pair(cp, m):
        c0 = cp * 2
        db, sb = load_desc(c0 + 1, dstb_b, srcb_b, semd_b, sems_b)
        db.start()
        sb.start()
        da, sa = load_desc(c0, dstb_a, srcb_a, semd_a, sems_a)
        with jax.named_scope("phase_ldwait"):
            da.wait()
            sa.wait()
        m = process_chunk(m, dstb_a, srcb_a)

        @pl.when(c0 + 2 < NCHUNK)
        def _():
            da2, sa2 = load_desc(c0 + 2, dstb_a, srcb_a, semd_a, sems_a)
            da2.start()
            sa2.start()

        with jax.named_scope("phase_ldwait"):
            db.wait()
            sb.wait()
        m = process_chunk(m, dstb_b, srcb_b)
        return m

    m = lax.fori_loop(0, NCHUNK // 2, chunk_pair, 0)

    for j in range(GK // L):
        ldb[pl.ds(m + j * L, L)] = padv
        srcc[pl.ds(m + j * L, L)] = zv

    @pl.when(m > 0)
    def _():
        gather_desc(0, brows0, semg0).start()
        gather_desc(0, brows0, semg0).wait()
        update(0, brows0)

    ng = jnp.minimum(NPT, NN - lo) // GR
    fslots = (brows0.at[pl.ds(0, GR)], brows1.at[pl.ds(0, GR)])
    fsems = (semg0, semg1)

    def fin_desc(g, slot):
        return pltpu.make_async_copy(
            a_hbm.at[pl.ds(lo + g * GR, GR)], fslots[slot], fsems[slot])

    with jax.named_scope("phase_fin"):
        fin_desc(0, 0).start()
        for g in range(NPT // GR):
            @pl.when(g < ng)
            def _(g=g):
                abuf = fslots[g % 2]
                fin_desc(g, g % 2).wait()
                if g + 1 < NPT // GR:
                    @pl.when(g + 1 < ng)
                    def _():
                        fin_desc(g + 1, (g + 1) % 2).start()

                def row_fin(r, _):
                    row0 = g * GR
                    for f in range(NF // 2):
                        cw = cbufs[f][pl.ds((row0 + r) * L, L)]
                        c0 = plsc.bitcast(cw << 16, jnp.float32)
                        c1 = plsc.bitcast(cw & hi_mask, jnp.float32)
                        s0 = pl.ds(f * 2 * L, L)
                        s1 = pl.ds(f * 2 * L + L, L)
                        abuf[r, s0] = jnp.maximum(abuf[r, s0] + c0, 0.0)
                        abuf[r, s1] = jnp.maximum(abuf[r, s1] + c1, 0.0)
                    return 0

                lax.fori_loop(0, GR, row_fin, 0, unroll=2)
                pltpu.sync_copy(abuf, out_hbm.at[pl.ds(lo + g * GR, GR)])

    return None


def kernel(x, edge_index, W, b):
    b8 = jnp.broadcast_to(b.reshape(1, D), (8, D))
    a, bb = _precompute(x, W, b8)
    src = edge_index[0]
    dst = edge_index[1]
    return _sc_segmax(a, bb, dst, src)


# speedup vs baseline: 1.0700x; 1.0434x over previous
import functools

import jax
import jax.numpy as jnp
from jax import lax
from jax.experimental import pallas as pl
from jax.experimental.pallas import tpu as pltpu
from jax.experimental.pallas import tpu_sc as plsc

NN = 10000
D = 128
NE = 320000

NC = 2
NS = 16
L = 16
NW = NC * NS
NPT = 320
NF = D // L
CH = 8000
NCHUNK = NE // CH
GK = 128
GR = 80


def _mm_body(x_ref, w_ref, b_ref, a_ref, bb_ref):
    x = x_ref[...]
    w1 = w_ref[:D, :]
    w2 = w_ref[D:, :]
    bb = jnp.dot(x, w2, preferred_element_type=jnp.float32)
    bb_ref[...] = bb
    a_ref[...] = jnp.dot(x, w1 - w2, preferred_element_type=jnp.float32) + b_ref[0:1, :]


def _precompute(x, W, b8):
    return pl.pallas_call(
        _mm_body,
        grid=(10,),
        in_specs=[
            pl.BlockSpec((1000, D), lambda i: (i, 0)),
            pl.BlockSpec((2 * D, D), lambda i: (0, 0)),
            pl.BlockSpec((8, D), lambda i: (0, 0)),
        ],
        out_specs=[
            pl.BlockSpec((1000, D), lambda i: (i, 0)),
            pl.BlockSpec((1000, D), lambda i: (i, 0)),
        ],
        out_shape=[
            jax.ShapeDtypeStruct((NN, D), jnp.float32),
            jax.ShapeDtypeStruct((NN, D), jnp.float32),
        ],
    )(x, W, b8)


@functools.partial(
    pl.kernel,
    out_type=jax.ShapeDtypeStruct((NN, D), jnp.float32),
    mesh=plsc.VectorSubcoreMesh(core_axis_name="c", subcore_axis_name="s"),
    compiler_params=pltpu.CompilerParams(needs_layout_passes=False),
    scratch_types=[
        [pltpu.VMEM(((NPT + 1) * L,), jnp.int32) for _ in range(NF // 2)],
        pltpu.VMEM((CH,), jnp.int32),
        pltpu.VMEM((CH,), jnp.int32),
        pltpu.VMEM((CH,), jnp.int32),
        pltpu.VMEM((CH,), jnp.int32),
        pltpu.VMEM((CH + GK,), jnp.int32),
        pltpu.VMEM((CH + GK,), jnp.int32),
        pltpu.VMEM((GK, D), jnp.float32),
        pltpu.VMEM((GK, D), jnp.float32),
        pltpu.SemaphoreType.DMA,
        pltpu.SemaphoreType.DMA,
        pltpu.SemaphoreType.DMA,
        pltpu.SemaphoreType.DMA,
        pltpu.SemaphoreType.DMA,
        pltpu.SemaphoreType.DMA,
    ],
)
def _sc_segmax(a_hbm, b_hbm, dst_hbm, src_hbm, out_hbm,
               cbufs, dstb_a, dstb_b, srcb_a, srcb_b, ldb, srcc,
               brows0, brows1, semd_a, semd_b, sems_a, sems_b, semg0, semg1):
    wid = lax.axis_index("s") * NC + lax.axis_index("c")
    lo = wid * NPT

    neg = jnp.full((L,), jnp.uint32(0xFF80FF80).astype(jnp.int32), jnp.int32)

    def init_body(i, _):
        for f in range(NF // 2):
            cbufs[f][pl.ds(i * L, L)] = neg
        return 0

    lax.fori_loop(0, NPT + 1, init_body, 0, unroll=2)

    padv = jnp.full((L,), NPT * L, jnp.int32)
    zv = jnp.zeros((L,), jnp.int32)

    def load_desc(ci, dstb, srcb, semd, sems):
        dd = pltpu.make_async_copy(dst_hbm.at[pl.ds(ci * CH, CH)], dstb, semd)
        ss = pltpu.make_async_copy(src_hbm.at[pl.ds(ci * CH, CH)], srcb, sems)
        return dd, ss

    def gather_desc(k0, brows, semg):
        return pltpu.make_async_copy(b_hbm.at[srcc.at[pl.ds(k0, GK)]], brows, semg)

    hi_mask = jnp.full((L,), jnp.uint32(0xFFFF0000).astype(jnp.int32), jnp.int32)

    def update(k0, brows):
        def egroup_body(eg, _):
            bases = ldb[pl.ds(k0 + eg * L, L)]
            for j in range(L):
                base = bases[j]
                row = eg * L + j
                sl = pl.ds(base, L)
                for f in range(NF // 2):
                    b0 = brows[row, pl.ds(f * 2 * L, L)]
                    b1 = brows[row, pl.ds(f * 2 * L + L, L)]
                    cb = cbufs[f]
                    cw = cb[sl]
                    c0 = plsc.bitcast(cw << 16, jnp.float32)
                    c1 = plsc.bitcast(cw & hi_mask, jnp.float32)
                    m0 = plsc.bitcast(jnp.maximum(c0, b0), jnp.int32)
                    m1 = plsc.bitcast(jnp.maximum(c1, b1), jnp.int32)
                    cb[sl] = jnp.bitwise_or(
                        lax.shift_right_logical(m0, 16), m1 & hi_mask)
            return 0

        with jax.named_scope("phase_upd"):
            lax.fori_loop(0, GK // L, egroup_body, 0)

    def process_chunk(m, dstb, srcb):
        def filt(gi, mm):
            d = dstb[pl.ds(gi * L, L)]
            s = srcb[pl.ds(gi * L, L)]
            t = d - lo
            msk = plsc.bitcast(t, jnp.uint32) < jnp.uint32(NPT)
            cs = plsc.cumsum(msk.astype(jnp.int32))
            pos = mm + cs - 1
            plsc.store_scatter(ldb, [pos], t * L, mask=msk)
            plsc.store_scatter(srcc, [pos], s, mask=msk)
            return mm + cs[L - 1]

        with jax.named_scope("phase_filt"):
            m = plsc.parallel_loop(0, CH // L, carry=m, unroll=8)(filt)

        nr = m >> 7

        def pair_round(rp, _):
            r0 = rp * 2

            @pl.when(r0 + 1 < nr)
            def _():
                gather_desc((r0 + 1) * GK, brows1, semg1).start()

            with jax.named_scope("phase_gwait"):
                gather_desc(r0 * GK, brows0, semg0).wait()
            update(r0 * GK, brows0)

            @pl.when(r0 + 2 < nr)
            def _():
                gather_desc((r0 + 2) * GK, brows0, semg0).start()

            @pl.when(r0 + 1 < nr)
            def _():
                with jax.named_scope("phase_gwait"):
                    gather_desc((r0 + 1) * GK, brows1, semg1).wait()
                update((r0 + 1) * GK, brows1)

            return 0

        with jax.named_scope("phase_rounds"):
            @pl.when(nr > 0)
            def _():
                gather_desc(0, brows0, semg0).start()

            lax.fori_loop(0, (nr + 1) >> 1, pair_round, 0)

            @pl.when(nr > 0)
            def _():
                for j in range(GK // L):
                    lv = ldb[pl.ds((nr << 7) + j * L, L)]
                    sv = srcc[pl.ds((nr << 7) + j * L, L)]
                    ldb[pl.ds(j * L, L)] = lv
                    srcc[pl.ds(j * L, L)] = sv

        return m - (nr << 7)

    da0, sa0 = load_desc(0, dstb_a, srcb_a, semd_a, sems_a)
    da0.start()
    sa0.start()

    def chunk_pair(cp, m):
        c0 = cp * 2
        db, sb = load_desc(c0 + 1, dstb_b, srcb_b, semd_b, sems_b)
        db.start()
        sb.start()
        da, sa = load_desc(c0, dstb_a, srcb_a, semd_a, sems_a)
        with jax.named_scope("phase_ldwait"):
            da.wait()
            sa.wait()
        m = process_chunk(m, dstb_a, srcb_a)

        @pl.when(c0 + 2 < NCHUNK)
        def _():
            da2, sa2 = load_desc(c0 + 2, dstb_a, srcb_a, semd_a, sems_a)
            da2.start()
            sa2.start()

        with jax.named_scope("phase_ldwait"):
            db.wait()
            sb.wait()
        m = process_chunk(m, dstb_b, srcb_b)
        return m

    m = lax.fori_loop(0, NCHUNK // 2, chunk_pair, 0)

    for j in range(GK // L):
        ldb[pl.ds(m + j * L, L)] = padv
        srcc[pl.ds(m + j * L, L)] = zv

    @pl.when(m > 0)
    def _():
        gather_desc(0, brows0, semg0).start()
        gather_desc(0, brows0, semg0).wait()
        update(0, brows0)

    ng = jnp.minimum(NPT, NN - lo) // GR
    fslots = (brows0.at[pl.ds(0, GR)], brows1.at[pl.ds(0, GR)])
    fsems = (semg0, semg1)

    def fin_desc(g, slot):
        return pltpu.make_async_copy(
            a_hbm.at[pl.ds(lo + g * GR, GR)], fslots[slot], fsems[slot])

    with jax.named_scope("phase_fin"):
        fin_desc(0, 0).start()
        for g in range(NPT // GR):
            @pl.when(g < ng)
            def _(g=g):
                abuf = fslots[g % 2]
                fin_desc(g, g % 2).wait()
                if g + 1 < NPT // GR:
                    @pl.when(g + 1 < ng)
                    def _():
                        fin_desc(g + 1, (g + 1) % 2).start()

                def row_fin(r, _):
                    row0 = g * GR
                    for f in range(NF // 2):
                        cw = cbufs[f][pl.ds((row0 + r) * L, L)]
                        c0 = plsc.bitcast(cw << 16, jnp.float32)
                        c1 = plsc.bitcast(cw & hi_mask, jnp.float32)
                        s0 = pl.ds(f * 2 * L, L)
                        s1 = pl.ds(f * 2 * L + L, L)
                        abuf[r, s0] = jnp.maximum(abuf[r, s0] + c0, 0.0)
                        abuf[r, s1] = jnp.maximum(abuf[r, s1] + c1, 0.0)
                    return 0

                lax.fori_loop(0, GR, row_fin, 0, unroll=2)
                pltpu.sync_copy(abuf, out_hbm.at[pl.ds(lo + g * GR, GR)])

    return None


def kernel(x, edge_index, W, b):
    b8 = jnp.broadcast_to(b.reshape(1, D), (8, D))
    a, bb = _precompute(x, W, b8)
    src = edge_index[0]
    dst = edge_index[1]
    return _sc_segmax(a, bb, dst, src)
